# baseline (speedup 1.0000x reference)
import jax
import jax.numpy as jnp
from jax import lax
from jax.experimental import pallas as pl
from jax.experimental.pallas import tpu as pltpu


def _body(idx_ref, tbl_ref, out_ref):
    i = pl.program_id(0)
    dim, vocab = tbl_ref.shape
    _, _, height, width = out_ref.shape
    v = idx_ref[i]
    sel = lax.broadcasted_iota(jnp.int32, (dim, vocab), 1) == v
    col = jnp.sum(jnp.where(sel, tbl_ref[...], 0.0), axis=1, keepdims=True)
    out_ref[...] = jnp.broadcast_to(col.reshape(1, dim, 1, 1),
                                    (1, dim, height, width))


def kernel(spatialPos, numFrames, frameIndices, frameEmbed):
    _, _, height, width = spatialPos.shape
    n_frames = frameIndices.shape[0]
    vocab, dim = frameEmbed.shape

    grid_spec = pltpu.PrefetchScalarGridSpec(
        num_scalar_prefetch=1,
        grid=(n_frames,),
        in_specs=[pl.BlockSpec((dim, vocab), lambda i, s: (0, 0))],
        out_specs=pl.BlockSpec((1, dim, height, width),
                               lambda i, s: (i, 0, 0, 0)),
    )
    return pl.pallas_call(
        _body,
        grid_spec=grid_spec,
        out_shape=jax.ShapeDtypeStruct((n_frames, dim, height, width),
                                       jnp.float32),
    )(frameIndices.astype(jnp.int32), frameEmbed.T)
